# Initial kernel scaffold; baseline (speedup 1.0000x reference)
import jax
import jax.numpy as jnp
from jax.experimental import pallas as pl


def kernel(hidden_states, W1, b1, W2, b2):
    raise NotImplementedError("write your pallas kernel here")


# speedup vs baseline: 4.5299x; 4.5299x over previous
import functools

import jax
import jax.numpy as jnp
from jax.experimental import pallas as pl

_TOPK = 2
_CAP_FACTOR = 1.5


def _router_body(x_ref, w1_ref, b1_ref, w2_ref, b2_ref,
                 probs_ref, meta_ref, pres_ref, psum_ref):
    i = pl.program_id(0)
    x = x_ref[...]
    h = jnp.maximum(
        jnp.dot(x, w1_ref[...], preferred_element_type=jnp.float32)
        + b1_ref[...], 0.0)
    logits = (jnp.dot(h, w2_ref[...], preferred_element_type=jnp.float32)
              + b2_ref[...])
    m = jnp.max(logits, axis=-1, keepdims=True)
    ex = jnp.exp(logits - m)
    probs = ex / jnp.sum(ex, axis=-1, keepdims=True)
    probs_ref[...] = probs

    tn, e = probs.shape
    eio = jax.lax.broadcasted_iota(jnp.int32, (tn, e), 1).astype(jnp.float32)
    p0 = jnp.max(probs, axis=-1, keepdims=True)
    e0 = jnp.min(jnp.where(probs == p0, eio, float(e)), axis=-1, keepdims=True)
    masked = jnp.where(eio == e0, -1.0, probs)
    p1 = jnp.max(masked, axis=-1, keepdims=True)
    e1 = jnp.min(jnp.where(masked == p1, eio, float(e)), axis=-1, keepdims=True)
    s = p0 + p1
    pad = jnp.zeros((tn, 4), jnp.float32)
    meta_ref[...] = jnp.concatenate([e0, e1, p0 / s, p1 / s, pad], axis=1)

    pres = jnp.max((eio == e0).astype(jnp.float32), axis=0, keepdims=True)
    psum = jnp.sum(probs, axis=0, keepdims=True)

    @pl.when(i == 0)
    def _():
        pres_ref[...] = pres
        psum_ref[...] = psum

    @pl.when(i > 0)
    def _():
        pres_ref[...] = jnp.maximum(pres_ref[...], pres)
        psum_ref[...] = psum_ref[...] + psum


def _scatter_body(meta_ref, pres_ref, psum_ref,
                  disp_ref, comb_ref, aux_ref, *, cap, n_tokens):
    i = pl.program_id(0)
    meta = meta_ref[...]
    tm = meta.shape[0]
    pres = pres_ref[...]
    n_e = pres.shape[1]
    e0 = meta[:, 0].reshape(tm, 1, 1)
    e1 = meta[:, 1].reshape(tm, 1, 1)
    p0 = meta[:, 2].reshape(tm, 1, 1)
    p1 = meta[:, 3].reshape(tm, 1, 1)
    eio = jax.lax.broadcasted_iota(jnp.int32, (tm, n_e), 1).astype(jnp.float32)
    oh1 = (eio == meta[:, 1].reshape(tm, 1)).astype(jnp.float32)
    pos1 = jnp.sum(oh1 * pres, axis=1).reshape(tm, 1, 1)

    e_ax = jax.lax.broadcasted_iota(
        jnp.int32, (tm, n_e, cap), 1).astype(jnp.float32)
    c_ax = jax.lax.broadcasted_iota(
        jnp.int32, (tm, n_e, cap), 2).astype(jnp.float32)
    hit0 = (e_ax == e0) & (c_ax == 0.0)
    hit1 = (e_ax == e1) & (c_ax == pos1)
    disp_ref[...] = hit0.astype(jnp.float32) + hit1.astype(jnp.float32)
    comb_ref[...] = jnp.where(hit0, p0, 0.0) + jnp.where(hit1, p1, 0.0)

    @pl.when(i == 0)
    def _():
        mean_p = psum_ref[...] * (1.0 / n_tokens)
        aux = jnp.sum(mean_p * jnp.log(mean_p * n_e + 1e-9))
        aux_ref[...] = aux.reshape(1, 1)


def kernel(hidden_states, W1, b1, W2, b2):
    b, s, h = hidden_states.shape
    e = W2.shape[1]
    n = b * s
    cap = int(b * s * _CAP_FACTOR * _TOPK / e)
    x = hidden_states.reshape(n, h)

    tb = 256
    probs, meta, pres, psum = pl.pallas_call(
        _router_body,
        grid=(n // tb,),
        in_specs=[
            pl.BlockSpec((tb, h), lambda i: (i, 0)),
            pl.BlockSpec((h, h), lambda i: (0, 0)),
            pl.BlockSpec((1, h), lambda i: (0, 0)),
            pl.BlockSpec((h, e), lambda i: (0, 0)),
            pl.BlockSpec((1, e), lambda i: (0, 0)),
        ],
        out_specs=[
            pl.BlockSpec((tb, e), lambda i: (i, 0)),
            pl.BlockSpec((tb, 8), lambda i: (i, 0)),
            pl.BlockSpec((1, e), lambda i: (0, 0)),
            pl.BlockSpec((1, e), lambda i: (0, 0)),
        ],
        out_shape=[
            jax.ShapeDtypeStruct((n, e), jnp.float32),
            jax.ShapeDtypeStruct((n, 8), jnp.float32),
            jax.ShapeDtypeStruct((1, e), jnp.float32),
            jax.ShapeDtypeStruct((1, e), jnp.float32),
        ],
    )(x, W1, b1.reshape(1, h), W2, b2.reshape(1, e))

    tm = 128
    disp, comb, aux = pl.pallas_call(
        functools.partial(_scatter_body, cap=cap, n_tokens=n),
        grid=(n // tm,),
        in_specs=[
            pl.BlockSpec((tm, 8), lambda i: (i, 0)),
            pl.BlockSpec((1, e), lambda i: (0, 0)),
            pl.BlockSpec((1, e), lambda i: (0, 0)),
        ],
        out_specs=[
            pl.BlockSpec((tm, e, cap), lambda i: (i, 0, 0)),
            pl.BlockSpec((tm, e, cap), lambda i: (i, 0, 0)),
            pl.BlockSpec((1, 1), lambda i: (0, 0)),
        ],
        out_shape=[
            jax.ShapeDtypeStruct((n, e, cap), jnp.float32),
            jax.ShapeDtypeStruct((n, e, cap), jnp.float32),
            jax.ShapeDtypeStruct((1, 1), jnp.float32),
        ],
    )(meta, pres, psum)

    return (disp.reshape(b, s, e, cap), comb.reshape(b, s, e, cap),
            probs.reshape(b, s, e), aux[0, 0])


# speedup vs baseline: 4.9558x; 1.0940x over previous
import functools

import jax
import jax.numpy as jnp
from jax.experimental import pallas as pl

_TOPK = 2
_CAP_FACTOR = 1.5


def _router_body(x_ref, w1_ref, b1_ref, w2_ref, b2_ref,
                 probs_ref, meta_ref, pres_ref, psum_ref):
    i = pl.program_id(0)
    x = x_ref[...]
    h = jnp.maximum(
        jnp.dot(x, w1_ref[...], preferred_element_type=jnp.float32)
        + b1_ref[...], 0.0)
    logits = (jnp.dot(h, w2_ref[...], preferred_element_type=jnp.float32)
              + b2_ref[...])
    m = jnp.max(logits, axis=-1, keepdims=True)
    ex = jnp.exp(logits - m)
    probs = ex / jnp.sum(ex, axis=-1, keepdims=True)
    probs_ref[...] = probs

    tn, e = probs.shape
    eio = jax.lax.broadcasted_iota(jnp.int32, (tn, e), 1).astype(jnp.float32)
    p0 = jnp.max(probs, axis=-1, keepdims=True)
    e0 = jnp.min(jnp.where(probs == p0, eio, float(e)), axis=-1, keepdims=True)
    masked = jnp.where(eio == e0, -1.0, probs)
    p1 = jnp.max(masked, axis=-1, keepdims=True)
    e1 = jnp.min(jnp.where(masked == p1, eio, float(e)), axis=-1, keepdims=True)
    s = p0 + p1
    pad = jnp.zeros((tn, 4), jnp.float32)
    meta_ref[...] = jnp.concatenate([e0, e1, p0 / s, p1 / s, pad], axis=1)

    pres = jnp.max((eio == e0).astype(jnp.float32), axis=0, keepdims=True)
    psum = jnp.sum(probs, axis=0, keepdims=True)

    @pl.when(i == 0)
    def _():
        pres_ref[...] = pres
        psum_ref[...] = psum

    @pl.when(i > 0)
    def _():
        pres_ref[...] = jnp.maximum(pres_ref[...], pres)
        psum_ref[...] = psum_ref[...] + psum


def _scatter_body(meta_ref, pres_ref, psum_ref,
                  disp_ref, comb_ref, aux_ref, *, cap, n_tokens):
    i = pl.program_id(0)
    meta = meta_ref[...]
    tm = meta.shape[0]
    pres = pres_ref[...]
    n_e = pres.shape[1]
    e0 = meta[:, 0].reshape(tm, 1, 1)
    e1 = meta[:, 1].reshape(tm, 1, 1)
    p0 = meta[:, 2].reshape(tm, 1, 1)
    p1 = meta[:, 3].reshape(tm, 1, 1)
    eio = jax.lax.broadcasted_iota(jnp.int32, (tm, n_e), 1).astype(jnp.float32)
    oh1 = (eio == meta[:, 1].reshape(tm, 1)).astype(jnp.float32)
    pos1 = jnp.sum(oh1 * pres, axis=1).reshape(tm, 1, 1)

    w = 128
    e_ax = jax.lax.broadcasted_iota(
        jnp.int32, (tm, n_e, w), 1).astype(jnp.float32)
    c_ax = jax.lax.broadcasted_iota(
        jnp.int32, (tm, n_e, w), 2).astype(jnp.float32)
    hit0 = (e_ax == e0) & (c_ax == 0.0)
    hit1 = (e_ax == e1) & (c_ax == pos1)
    disp_ref[:, :, 0:w] = hit0.astype(jnp.float32) + hit1.astype(jnp.float32)
    comb_ref[:, :, 0:w] = (jnp.where(hit0, p0, 0.0)
                           + jnp.where(hit1, p1, 0.0))
    zt = jnp.zeros((tm, n_e, cap - w), jnp.float32)
    disp_ref[:, :, w:cap] = zt
    comb_ref[:, :, w:cap] = zt

    @pl.when(i == 0)
    def _():
        mean_p = psum_ref[...] * (1.0 / n_tokens)
        aux = jnp.sum(mean_p * jnp.log(mean_p * n_e + 1e-9))
        aux_ref[...] = aux.reshape(1, 1)


def kernel(hidden_states, W1, b1, W2, b2):
    b, s, h = hidden_states.shape
    e = W2.shape[1]
    n = b * s
    cap = int(b * s * _CAP_FACTOR * _TOPK / e)
    x = hidden_states.reshape(n, h)

    tb = 256
    probs, meta, pres, psum = pl.pallas_call(
        _router_body,
        grid=(n // tb,),
        in_specs=[
            pl.BlockSpec((tb, h), lambda i: (i, 0)),
            pl.BlockSpec((h, h), lambda i: (0, 0)),
            pl.BlockSpec((1, h), lambda i: (0, 0)),
            pl.BlockSpec((h, e), lambda i: (0, 0)),
            pl.BlockSpec((1, e), lambda i: (0, 0)),
        ],
        out_specs=[
            pl.BlockSpec((tb, e), lambda i: (i, 0)),
            pl.BlockSpec((tb, 8), lambda i: (i, 0)),
            pl.BlockSpec((1, e), lambda i: (0, 0)),
            pl.BlockSpec((1, e), lambda i: (0, 0)),
        ],
        out_shape=[
            jax.ShapeDtypeStruct((n, e), jnp.float32),
            jax.ShapeDtypeStruct((n, 8), jnp.float32),
            jax.ShapeDtypeStruct((1, e), jnp.float32),
            jax.ShapeDtypeStruct((1, e), jnp.float32),
        ],
    )(x, W1, b1.reshape(1, h), W2, b2.reshape(1, e))

    tm = 128
    disp, comb, aux = pl.pallas_call(
        functools.partial(_scatter_body, cap=cap, n_tokens=n),
        grid=(n // tm,),
        in_specs=[
            pl.BlockSpec((tm, 8), lambda i: (i, 0)),
            pl.BlockSpec((1, e), lambda i: (0, 0)),
            pl.BlockSpec((1, e), lambda i: (0, 0)),
        ],
        out_specs=[
            pl.BlockSpec((tm, e, cap), lambda i: (i, 0, 0)),
            pl.BlockSpec((tm, e, cap), lambda i: (i, 0, 0)),
            pl.BlockSpec((1, 1), lambda i: (0, 0)),
        ],
        out_shape=[
            jax.ShapeDtypeStruct((n, e, cap), jnp.float32),
            jax.ShapeDtypeStruct((n, e, cap), jnp.float32),
            jax.ShapeDtypeStruct((1, 1), jnp.float32),
        ],
    )(meta, pres, psum)

    return (disp.reshape(b, s, e, cap), comb.reshape(b, s, e, cap),
            probs.reshape(b, s, e), aux[0, 0])
